# Initial kernel scaffold; baseline (speedup 1.0000x reference)
import jax
import jax.numpy as jnp
from jax.experimental import pallas as pl


def kernel(sequence, token_table):
    raise NotImplementedError("write your pallas kernel here")


# speedup vs baseline: 2.4234x; 2.4234x over previous
import functools

import numpy as np
import jax
import jax.numpy as jnp
from jax import lax
from jax.experimental import pallas as pl
from jax.experimental.pallas import tpu as pltpu
from jax.experimental.pallas import tpu_sc as plsc

_NC = 2
_NS = 16
_NW = _NC * _NS
_LANES = 16


def _pe_table(max_len: int, d_model: int) -> np.ndarray:
    pe = np.zeros((max_len, d_model), dtype=np.float32)
    position = np.arange(max_len, dtype=np.float32)[:, None]
    div_term = np.exp(
        np.arange(0, d_model, 2, dtype=np.float32) * -(np.log(10000.0) / d_model)
    )
    pe[:, 0::2] = np.sin(position * div_term)
    pe[:, 1::2] = np.cos(position * div_term)
    return pe


@functools.lru_cache(maxsize=None)
def _build(B: int, S: int, V: int, E: int):
    total = B * S
    assert total % _NW == 0
    per_w = total // _NW
    assert per_w % S == 0
    seqs_per_w = per_w // S
    cs = 2 if seqs_per_w % 2 == 0 else 1
    chunk_rows = cs * S
    nchunks = seqs_per_w // cs
    subs = []
    off = 0
    while off < chunk_rows:
        ln = min(128, chunk_rows - off)
        subs.append((off, ln))
        off += ln

    mesh = plsc.VectorSubcoreMesh(
        core_axis_name="c", subcore_axis_name="s",
        num_cores=_NC, num_subcores=_NS,
    )

    @functools.partial(
        pl.kernel,
        out_type=jax.ShapeDtypeStruct((total, E), jnp.float32),
        mesh=mesh,
        compiler_params=pltpu.CompilerParams(use_tc_tiling_on_sc=False),
        scratch_types=[
            pltpu.VMEM((chunk_rows,), jnp.int32),
            pltpu.VMEM((chunk_rows, E), jnp.float32),
            pltpu.VMEM((S, E), jnp.float32),
            pltpu.SemaphoreType.DMA,
        ],
    )
    def emb(seq_hbm, table_hbm, pe_hbm, out_hbm, idx_v, rows_v, pe_v, sem):
        wid = lax.axis_index("s") * _NC + lax.axis_index("c")
        pltpu.sync_copy(pe_hbm, pe_v)

        def chunk_body(ci, carry):
            base = wid * per_w + ci * chunk_rows
            pltpu.sync_copy(seq_hbm.at[pl.ds(base, chunk_rows)], idx_v)
            cps = [
                pltpu.async_copy(
                    table_hbm.at[idx_v.at[pl.ds(off, ln)]],
                    rows_v.at[pl.ds(off, ln)],
                    sem,
                )
                for off, ln in subs
            ]
            for cp in cps:
                cp.wait()

            def add_body(r, c2):
                for k in range(cs):
                    row = k * S + r
                    for c in range(E // _LANES):
                        sl = pl.ds(c * _LANES, _LANES)
                        rows_v[row, sl] = rows_v[row, sl] + pe_v[r, sl]
                return c2

            lax.fori_loop(0, S, add_body, 0, unroll=2)
            pltpu.sync_copy(rows_v, out_hbm.at[pl.ds(base, chunk_rows)])
            return carry

        lax.fori_loop(0, nchunks, chunk_body, 0)

    pe_host = jnp.asarray(_pe_table(S, E))

    def run(sequence, token_table):
        out = emb(sequence.reshape(total), token_table, pe_host)
        return out.reshape(B, S, E)

    return run


def kernel(sequence, token_table):
    B, S = sequence.shape
    V, E = token_table.shape
    return _build(B, S, V, E)(sequence, token_table)


# speedup vs baseline: 2.9091x; 1.2004x over previous
import functools

import numpy as np
import jax
import jax.numpy as jnp
from jax import lax
from jax.experimental import pallas as pl
from jax.experimental.pallas import tpu as pltpu
from jax.experimental.pallas import tpu_sc as plsc

_NC = 2
_NS = 16
_NW = _NC * _NS
_LANES = 16
_RING = 4
_FD = 2


def _pe_table(max_len: int, d_model: int) -> np.ndarray:
    pe = np.zeros((max_len, d_model), dtype=np.float32)
    position = np.arange(max_len, dtype=np.float32)[:, None]
    div_term = np.exp(
        np.arange(0, d_model, 2, dtype=np.float32) * -(np.log(10000.0) / d_model)
    )
    pe[:, 0::2] = np.sin(position * div_term)
    pe[:, 1::2] = np.cos(position * div_term)
    return pe


@functools.lru_cache(maxsize=None)
def _build(B: int, S: int, V: int, E: int):
    assert B % _NW == 0 and E % _LANES == 0
    seqs_per_w = B // _NW
    assert seqs_per_w % _RING == 0 and seqs_per_w // _RING >= 2
    n_outer = seqs_per_w // _RING
    subs = []
    off = 0
    while off < S:
        ln = min(128, S - off)
        subs.append((off, ln))
        off += ln

    mesh = plsc.VectorSubcoreMesh(
        core_axis_name="c", subcore_axis_name="s",
        num_cores=_NC, num_subcores=_NS,
    )

    @functools.partial(
        pl.kernel,
        out_type=jax.ShapeDtypeStruct((B, S, E), jnp.float32),
        mesh=mesh,
        compiler_params=pltpu.CompilerParams(use_tc_tiling_on_sc=False),
        scratch_types=[
            pltpu.VMEM((seqs_per_w, S), jnp.int32),
            [pltpu.VMEM((S, E), jnp.float32) for _ in range(_RING)],
            pltpu.VMEM((S, E), jnp.float32),
            [pltpu.SemaphoreType.DMA for _ in range(2 * _RING)],
        ],
    )
    def emb(seq_hbm, table_hbm, pe_hbm, out_hbm, idx_v, rows, pe_v, sems):
        sg, so = sems[:_RING], sems[_RING:]
        wid = lax.axis_index("s") * _NC + lax.axis_index("c")
        seq0 = wid * seqs_per_w
        pltpu.sync_copy(pe_hbm, pe_v)
        pltpu.sync_copy(seq_hbm.at[pl.ds(seq0, seqs_per_w)], idx_v)

        def fire_gather(gl, b):
            for o, ln in subs:
                pltpu.async_copy(
                    table_hbm.at[idx_v.at[gl, pl.ds(o, ln)]],
                    rows[b].at[pl.ds(o, ln)],
                    sg[b],
                )

        def wait_gather(b):
            pltpu.make_async_copy(table_hbm.at[pl.ds(0, S)], rows[b], sg[b]).wait()

        def fire_out(gl, b):
            pltpu.async_copy(rows[b], out_hbm.at[seq0 + gl], so[b])

        def wait_out(gl_prev, b):
            pltpu.make_async_copy(rows[b], out_hbm.at[seq0 + gl_prev], so[b]).wait()

        def add_pe(b):
            rb = rows[b]

            def body(r, c2):
                for c in range(E // _LANES):
                    sl = pl.ds(c * _LANES, _LANES)
                    rb[r, sl] = rb[r, sl] + pe_v[r, sl]
                return c2

            lax.fori_loop(0, S, body, 0, unroll=8)

        def step(gl, b, do_fire, do_wait_out):
            wait_gather(b)
            nb = (b + _FD) % _RING
            if do_wait_out:
                wait_out(gl - _FD, nb)
            if do_fire:
                fire_gather(gl + _FD, nb)
            add_pe(b)
            fire_out(gl, b)

        fire_gather(0, 0)
        fire_gather(1, 1)
        step(0, 0, True, False)
        step(1, 1, True, False)
        step(2, 2, True, True)
        step(3, 3, True, True)

        def outer_body(i, carry):
            g0 = i * _RING
            for b in range(_RING):
                step(g0 + b, b, True, True)
            return carry

        lax.fori_loop(1, n_outer - 1, outer_body, 0)

        gl_last = seqs_per_w - _RING
        step(gl_last + 0, 0, True, True)
        step(gl_last + 1, 1, True, True)
        step(gl_last + 2, 2, False, False)
        step(gl_last + 3, 3, False, False)
        for b in range(_RING):
            wait_out(gl_last + b, b)

    pe_host = jnp.asarray(_pe_table(S, E))

    def run(sequence, token_table):
        return emb(sequence, token_table, pe_host)

    return run


def kernel(sequence, token_table):
    B, S = sequence.shape
    V, E = token_table.shape
    return _build(B, S, V, E)(sequence, token_table)


# speedup vs baseline: 3.3758x; 1.1604x over previous
import functools

import numpy as np
import jax
import jax.numpy as jnp
from jax import lax
from jax.experimental import pallas as pl
from jax.experimental.pallas import tpu as pltpu
from jax.experimental.pallas import tpu_sc as plsc

_NC = 2
_NS = 16
_NW = _NC * _NS
_LANES = 16
_PADE = 128
_BLK = 8


def _pe_table(max_len: int, d_model: int) -> np.ndarray:
    pe = np.zeros((max_len, d_model), dtype=np.float32)
    position = np.arange(max_len, dtype=np.float32)[:, None]
    div_term = np.exp(
        np.arange(0, d_model, 2, dtype=np.float32) * -(np.log(10000.0) / d_model)
    )
    pe[:, 0::2] = np.sin(position * div_term)
    pe[:, 1::2] = np.cos(position * div_term)
    return pe


@functools.lru_cache(maxsize=None)
def _build(B: int, S: int, V: int, E: int):
    assert B % _NW == 0 and E % _LANES == 0
    total = B * S
    seqs_per_w = B // _NW
    blk_idx = _BLK * S
    subs = []
    off = 0
    while off < S:
        ln = min(128, S - off)
        subs.append((off, ln))
        off += ln

    mesh = plsc.VectorSubcoreMesh(
        core_axis_name="c", subcore_axis_name="s",
        num_cores=_NC, num_subcores=_NS,
    )

    @functools.partial(
        pl.kernel,
        out_type=jax.ShapeDtypeStruct((B, S, E), jnp.float32),
        mesh=mesh,
        compiler_params=pltpu.CompilerParams(use_tc_tiling_on_sc=True),
        scratch_types=[
            pltpu.VMEM((2 * blk_idx,), jnp.int32),
            [pltpu.VMEM((S, _PADE), jnp.float32) for _ in range(2)],
            [pltpu.VMEM((S, E), jnp.float32) for _ in range(2)],
            pltpu.VMEM((S * E,), jnp.float32),
            [pltpu.SemaphoreType.DMA for _ in range(4)],
        ],
    )
    def emb(seq_hbm, table_hbm, pe_hbm, out_hbm, idx_v, rbufs, obufs, pe_v, sems):
        sg, so = sems[:2], sems[2:]
        wid = lax.axis_index("s") * _NC + lax.axis_index("c")
        seq0 = wid * seqs_per_w
        pltpu.sync_copy(pe_hbm, pe_v)

        def stage_idx_block(g_first):
            slot_off = ((g_first // _BLK) % 2) * blk_idx
            pltpu.sync_copy(
                seq_hbm.at[pl.ds((seq0 + g_first) * S, blk_idx)],
                idx_v.at[pl.ds(slot_off, blk_idx)],
            )

        def idx_off(gl, o):
            return ((gl // _BLK) % 2) * blk_idx + (gl % _BLK) * S + o

        def fire_gather(gl, b):
            for o, ln in subs:
                pltpu.async_copy(
                    table_hbm.at[idx_v.at[pl.ds(idx_off(gl, o), ln)]],
                    rbufs[b].at[pl.ds(o, ln)],
                    sg[b],
                )

        def wait_gather(gl, b):
            for o, ln in subs:
                pltpu.make_async_copy(
                    table_hbm.at[idx_v.at[pl.ds(idx_off(gl, o), ln)]],
                    rbufs[b].at[pl.ds(o, ln)],
                    sg[b],
                ).wait()

        def fire_out(gl, b):
            pltpu.async_copy(obufs[b], out_hbm.at[seq0 + gl], so[b])

        def wait_out(gl_prev, b):
            pltpu.make_async_copy(
                obufs[b], out_hbm.at[seq0 + gl_prev], so[b]
            ).wait()

        def add_pe(b):
            rb = rbufs[b]
            ob = obufs[b]

            def body(r, c2):
                for c in range(E // _LANES):
                    sl = pl.ds(c * _LANES, _LANES)
                    ob[r, sl] = rb[r, sl] + pe_v[pl.ds(r * E + c * _LANES, _LANES)]
                return c2

            lax.fori_loop(0, S, body, 0, unroll=8)

        def step(gl, b, do_fire, do_wait_out):
            wait_gather(gl, b)
            if do_wait_out:
                wait_out(gl - 2, b)
            add_pe(b)
            fire_out(gl, b)
            if do_fire:
                if isinstance(gl, int):
                    if (gl + 2) % _BLK == 0:
                        stage_idx_block(gl + 2)
                else:

                    @pl.when((gl + 2) % _BLK == 0)
                    def _():
                        stage_idx_block(gl + 2)

                fire_gather(gl + 2, b)

        n = seqs_per_w
        stage_idx_block(0)
        fire_gather(0, 0)
        fire_gather(1, 1)
        step(0, 0, True, False)
        step(1, 1, True, False)

        def outer_body(i, carry):
            g0 = 2 + 2 * i
            step(g0, 0, True, True)
            step(g0 + 1, 1, True, True)
            return carry

        lax.fori_loop(0, (n - 4) // 2, outer_body, 0)

        step(n - 2, 0, False, True)
        step(n - 1, 1, False, True)
        wait_out(n - 2, 0)
        wait_out(n - 1, 1)

    pe_host = jnp.asarray(_pe_table(S, E).reshape(-1))

    def run(sequence, token_table):
        table_p = jnp.pad(token_table, ((0, 0), (0, _PADE - E)))
        return emb(sequence.reshape(total), table_p, pe_host)

    return run


def kernel(sequence, token_table):
    B, S = sequence.shape
    V, E = token_table.shape
    return _build(B, S, V, E)(sequence, token_table)
